# initial kernel scaffold (unmeasured)
import jax
import jax.numpy as jnp
from jax import lax
from jax.experimental import pallas as pl
from jax.experimental.pallas import tpu as pltpu

N_DEV = 16
SCALE = 0.08838834764831843
HQ = 8
DH = 128


def kernel(x, Wq, Wo, K_ext, V_ext):
    _, sq, d = x.shape
    _, skv, hq, dh = K_ext.shape
    assert (hq, dh, hq * dh) == (HQ, DH, d)

    xb = x.reshape(sq, d).astype(jnp.bfloat16)
    kb = K_ext.reshape(skv, d).astype(jnp.bfloat16)
    vb = V_ext.reshape(skv, d).astype(jnp.bfloat16)
    wqb = Wq.astype(jnp.bfloat16)
    wob = Wo.astype(jnp.bfloat16)

    def body(x_ref, wq_ref, wo_ref, k_ref, v_ref, out_ref,
             q_buf, o_buf, m_buf, l_buf, y_src, y_dst,
             send_sems, recv_sems, y_send_sem, y_recv_sem, credit_sem):
        my = lax.axis_index("i")
        left = lax.rem(my + N_DEV - 1, N_DEV)
        right = lax.rem(my + 1, N_DEV)

        barrier = pltpu.get_barrier_semaphore()
        for nbr in (left, right):
            pl.semaphore_signal(barrier, inc=1, device_id=(nbr,),
                                device_id_type=pl.DeviceIdType.MESH)
        pl.semaphore_wait(barrier, 2)

        q = jnp.dot(x_ref[...], wq_ref[...],
                    preferred_element_type=jnp.float32) * SCALE
        q_buf[0, :, :] = q.astype(jnp.bfloat16)
        o_buf[0, :, :] = jnp.zeros((sq, d), jnp.float32)
        m_buf[0, :, :] = jnp.full((sq, HQ), -1e30, jnp.float32)
        l_buf[0, :, :] = jnp.zeros((sq, HQ), jnp.float32)

        state_bufs = (q_buf, o_buf, m_buf, l_buf)

        for h in range(N_DEV):
            cur = h % 2
            nxt = (h + 1) % 2

            for hd in range(HQ):
                c0 = hd * DH
                q_h = q_buf[cur, :, c0:c0 + DH]
                k_h = k_ref[:, c0:c0 + DH]
                v_h = v_ref[:, c0:c0 + DH]
                s = lax.dot_general(q_h, k_h, (((1,), (1,)), ((), ())),
                                    preferred_element_type=jnp.float32)
                m_old = m_buf[cur, :, hd:hd + 1]
                l_old = l_buf[cur, :, hd:hd + 1]
                o_old = o_buf[cur, :, c0:c0 + DH]
                m_new = jnp.maximum(m_old, jnp.max(s, axis=1, keepdims=True))
                alpha = jnp.exp(m_old - m_new)
                p = jnp.exp(s - m_new)
                l_new = l_old * alpha + jnp.sum(p, axis=1, keepdims=True)
                pv = lax.dot_general(p.astype(jnp.bfloat16), v_h,
                                     (((1,), (0,)), ((), ())),
                                     preferred_element_type=jnp.float32)
                m_buf[cur, :, hd:hd + 1] = m_new
                l_buf[cur, :, hd:hd + 1] = l_new
                o_buf[cur, :, c0:c0 + DH] = o_old * alpha + pv

            if h < N_DEV - 1:
                if h >= 1:
                    pl.semaphore_wait(credit_sem, 1)
                rdmas = []
                for i, buf in enumerate(state_bufs):
                    r = pltpu.make_async_remote_copy(
                        src_ref=buf.at[cur],
                        dst_ref=buf.at[nxt],
                        send_sem=send_sems.at[i, cur],
                        recv_sem=recv_sems.at[i, nxt],
                        device_id=(right,),
                        device_id_type=pl.DeviceIdType.MESH,
                    )
                    r.start()
                    rdmas.append(r)
                for r in rdmas:
                    r.wait()
                pl.semaphore_signal(credit_sem, inc=1, device_id=(left,),
                                    device_id_type=pl.DeviceIdType.MESH)
            else:
                for hd in range(HQ):
                    c0 = hd * DH
                    o_buf[cur, :, c0:c0 + DH] = (
                        o_buf[cur, :, c0:c0 + DH] / l_buf[cur, :, hd:hd + 1])
                y = jnp.dot(o_buf[cur, :, :].astype(jnp.bfloat16), wo_ref[...],
                            preferred_element_type=jnp.float32)
                y_src[...] = y
                r = pltpu.make_async_remote_copy(
                    src_ref=y_src, dst_ref=y_dst,
                    send_sem=y_send_sem, recv_sem=y_recv_sem,
                    device_id=(right,), device_id_type=pl.DeviceIdType.MESH,
                )
                r.start()
                r.wait()
                out_ref[...] = y_dst[...]

    out = pl.pallas_call(
        body,
        out_shape=jax.ShapeDtypeStruct((sq, d), jnp.float32),
        in_specs=[pl.BlockSpec(memory_space=pltpu.VMEM)] * 5,
        out_specs=pl.BlockSpec(memory_space=pltpu.VMEM),
        scratch_shapes=[
            pltpu.VMEM((2, sq, d), jnp.bfloat16),
            pltpu.VMEM((2, sq, d), jnp.float32),
            pltpu.VMEM((2, sq, HQ), jnp.float32),
            pltpu.VMEM((2, sq, HQ), jnp.float32),
            pltpu.VMEM((sq, d), jnp.float32),
            pltpu.VMEM((sq, d), jnp.float32),
            pltpu.SemaphoreType.DMA((4, 2)),
            pltpu.SemaphoreType.DMA((4, 2)),
            pltpu.SemaphoreType.DMA,
            pltpu.SemaphoreType.DMA,
            pltpu.SemaphoreType.REGULAR,
        ],
        compiler_params=pltpu.CompilerParams(collective_id=0),
    )(xb, wqb, wob, kb, vb)
    return out.reshape(1, sq, d)


# baseline (device time: 650249 ns/iter reference)
import jax
import jax.numpy as jnp
from jax import lax
from jax.experimental import pallas as pl
from jax.experimental.pallas import tpu as pltpu

N_DEV = 16
SCALE = 0.08838834764831843
HQ = 8
DH = 128


def kernel(x, Wq, Wo, K_ext, V_ext):
    _, sq, d = x.shape
    _, skv, hq, dh = K_ext.shape
    assert (hq, dh, hq * dh) == (HQ, DH, d)

    q = jnp.dot(x.reshape(sq, d).astype(jnp.bfloat16), Wq.astype(jnp.bfloat16),
                preferred_element_type=jnp.float32) * SCALE
    q3 = q.reshape(sq, HQ, DH).transpose(1, 0, 2).reshape(HQ * sq, DH)
    q3 = q3.astype(jnp.bfloat16)
    k3 = K_ext.reshape(skv, HQ, DH).transpose(1, 0, 2).reshape(HQ * skv, DH)
    k3 = k3.astype(jnp.bfloat16)
    v3 = V_ext.reshape(skv, HQ, DH).transpose(1, 0, 2).reshape(HQ * skv, DH)
    v3 = v3.astype(jnp.bfloat16)

    def body(q_ref, k_ref, v_ref, out_ref,
             q_buf, o_buf, ml_buf, y_dst, c_src, c_dst,
             q_send, q_recv, o_send, o_recv, ml_send, ml_recv,
             y_send, y_recv, c_send, c_recv):
        my = lax.axis_index("i")
        left = lax.rem(my + N_DEV - 1, N_DEV)
        right = lax.rem(my + 1, N_DEV)

        def credit_rdma(slot):
            return pltpu.make_async_remote_copy(
                src_ref=c_src, dst_ref=c_dst,
                send_sem=c_send.at[slot], recv_sem=c_recv.at[slot],
                device_id=(left,), device_id_type=pl.DeviceIdType.MESH,
            )

        barrier = pltpu.get_barrier_semaphore()
        for nbr in (left, right):
            pl.semaphore_signal(barrier, inc=1, device_id=(nbr,),
                                device_id_type=pl.DeviceIdType.MESH)
        pl.semaphore_wait(barrier, 2)

        q_buf[0] = q_ref[...]
        o_buf[0] = jnp.zeros((HQ * DH, sq), jnp.float32)
        ml_buf[0, :HQ] = jnp.full((HQ, sq), -1e30, jnp.float32)
        ml_buf[0, HQ:] = jnp.zeros((HQ, sq), jnp.float32)

        for h in range(N_DEV):
            cur = h % 2
            nxt = (h + 1) % 2

            def head_step(hd, carry):
                q_h = q_buf[cur, pl.ds(hd * sq, sq), :]
                k_h = k_ref[pl.ds(hd * skv, skv), :]
                v_h = v_ref[pl.ds(hd * skv, skv), :]
                s_t = lax.dot_general(k_h, q_h, (((1,), (1,)), ((), ())),
                                      preferred_element_type=jnp.float32)
                m_old = ml_buf[cur, hd]
                l_old = ml_buf[cur, HQ + hd]
                o_old = o_buf[cur, pl.ds(hd * DH, DH), :]
                m_new = jnp.maximum(m_old, jnp.max(s_t, axis=0))
                alpha = jnp.exp(m_old - m_new)
                p_t = jnp.exp(s_t - m_new)
                l_new = l_old * alpha + jnp.sum(p_t, axis=0)
                pv_t = lax.dot_general(v_h, p_t.astype(jnp.bfloat16),
                                       (((0,), (0,)), ((), ())),
                                       preferred_element_type=jnp.float32)
                ml_buf[cur, hd] = m_new
                ml_buf[cur, HQ + hd] = l_new
                o_buf[cur, pl.ds(hd * DH, DH), :] = o_old * alpha + pv_t
                return carry

            lax.fori_loop(0, HQ, head_step, 0)

            if h < N_DEV - 1:
                if h >= 1:
                    credit_rdma(nxt).wait_recv()
                rdmas = []
                for buf, ssem, rsem in (
                    (q_buf, q_send, q_recv),
                    (o_buf, o_send, o_recv),
                    (ml_buf, ml_send, ml_recv),
                ):
                    r = pltpu.make_async_remote_copy(
                        src_ref=buf.at[cur],
                        dst_ref=buf.at[nxt],
                        send_sem=ssem.at[cur],
                        recv_sem=rsem.at[nxt],
                        device_id=(right,),
                        device_id_type=pl.DeviceIdType.MESH,
                    )
                    r.start()
                    rdmas.append(r)
                for r in rdmas:
                    r.wait()
                if h < N_DEV - 2:
                    cr = credit_rdma(cur)
                    cr.start()
                    cr.wait_send()
            else:
                def norm_step(hd, carry):
                    o_buf[cur, pl.ds(hd * DH, DH), :] = (
                        o_buf[cur, pl.ds(hd * DH, DH), :] / ml_buf[cur, HQ + hd])
                    return carry

                lax.fori_loop(0, HQ, norm_step, 0)
                r = pltpu.make_async_remote_copy(
                    src_ref=o_buf.at[cur], dst_ref=y_dst,
                    send_sem=y_send, recv_sem=y_recv,
                    device_id=(right,), device_id_type=pl.DeviceIdType.MESH,
                )
                r.start()
                r.wait()
                out_ref[...] = y_dst[...]

    attn = pl.pallas_call(
        body,
        out_shape=jax.ShapeDtypeStruct((HQ * DH, sq), jnp.float32),
        in_specs=[pl.BlockSpec(memory_space=pltpu.VMEM)] * 3,
        out_specs=pl.BlockSpec(memory_space=pltpu.VMEM),
        scratch_shapes=[
            pltpu.VMEM((2, HQ * sq, DH), jnp.bfloat16),
            pltpu.VMEM((2, HQ * DH, sq), jnp.float32),
            pltpu.VMEM((2, 2 * HQ, sq), jnp.float32),
            pltpu.VMEM((HQ * DH, sq), jnp.float32),
            pltpu.VMEM((8, 128), jnp.float32),
            pltpu.VMEM((8, 128), jnp.float32),
            pltpu.SemaphoreType.DMA((2,)),
            pltpu.SemaphoreType.DMA((2,)),
            pltpu.SemaphoreType.DMA((2,)),
            pltpu.SemaphoreType.DMA((2,)),
            pltpu.SemaphoreType.DMA((2,)),
            pltpu.SemaphoreType.DMA((2,)),
            pltpu.SemaphoreType.DMA,
            pltpu.SemaphoreType.DMA,
            pltpu.SemaphoreType.DMA((2,)),
            pltpu.SemaphoreType.DMA((2,)),
        ],
        compiler_params=pltpu.CompilerParams(collective_id=0),
    )(q3, k3, v3)

    y = attn.reshape(HQ, DH, sq).transpose(2, 0, 1).reshape(sq, d)
    out = jnp.dot(y.astype(jnp.bfloat16), Wo.astype(jnp.bfloat16),
                  preferred_element_type=jnp.float32)
    return out.reshape(1, sq, d)


# device time: 510159 ns/iter; 1.2746x vs baseline; 1.2746x over previous
import jax
import jax.numpy as jnp
from jax import lax
from jax.experimental import pallas as pl
from jax.experimental.pallas import tpu as pltpu

N_DEV = 16
SCALE = 0.08838834764831843
HQ = 8
DH = 128


def kernel(x, Wq, Wo, K_ext, V_ext):
    _, sq, d = x.shape
    _, skv, hq, dh = K_ext.shape
    assert (hq, dh, hq * dh) == (HQ, DH, d)

    q = jnp.dot(x.reshape(sq, d).astype(jnp.bfloat16), Wq.astype(jnp.bfloat16),
                preferred_element_type=jnp.float32) * SCALE
    q3 = q.reshape(sq, HQ, DH).transpose(1, 0, 2).reshape(HQ * sq, DH)
    q3 = q3.astype(jnp.bfloat16)
    k3 = K_ext.reshape(skv, HQ, DH).transpose(1, 0, 2).reshape(HQ * skv, DH)
    k3 = k3.astype(jnp.bfloat16)
    v3 = V_ext.reshape(skv, HQ, DH).transpose(1, 0, 2).reshape(HQ * skv, DH)
    v3 = v3.astype(jnp.bfloat16)

    def body(q_ref, k_ref, v_ref, out_ref,
             q_buf, o_buf, ml_buf, y_dst, c_src, c_dst,
             q_send, q_recv, o0_send, o0_recv, o1_send, o1_recv,
             ml_send, ml_recv, y_send, y_recv, c_send, c_recv):
        my = lax.axis_index("i")
        left = lax.rem(my + N_DEV - 1, N_DEV)
        right = lax.rem(my + 1, N_DEV)

        def credit_rdma(slot):
            return pltpu.make_async_remote_copy(
                src_ref=c_src, dst_ref=c_dst,
                send_sem=c_send.at[slot], recv_sem=c_recv.at[slot],
                device_id=(left,), device_id_type=pl.DeviceIdType.MESH,
            )

        barrier = pltpu.get_barrier_semaphore()
        for nbr in (left, right):
            pl.semaphore_signal(barrier, inc=1, device_id=(nbr,),
                                device_id_type=pl.DeviceIdType.MESH)
        pl.semaphore_wait(barrier, 2)

        q_buf[0] = q_ref[...]
        o_buf[0] = jnp.zeros((HQ * DH, sq), jnp.float32)
        ml_buf[0, :HQ] = jnp.full((HQ, sq), -1e30, jnp.float32)
        ml_buf[0, HQ:] = jnp.zeros((HQ, sq), jnp.float32)

        for h in range(N_DEV):
            cur = h % 2
            nxt = (h + 1) % 2

            def head_step(hd, carry):
                q_h = q_buf[cur, pl.ds(hd * sq, sq), :]
                k_h = k_ref[pl.ds(hd * skv, skv), :]
                v_h = v_ref[pl.ds(hd * skv, skv), :]
                s_t = lax.dot_general(k_h, q_h, (((1,), (1,)), ((), ())),
                                      preferred_element_type=jnp.float32)
                m_old = ml_buf[cur, hd]
                l_old = ml_buf[cur, HQ + hd]
                o_old = o_buf[cur, pl.ds(hd * DH, DH), :]
                m_new = jnp.maximum(m_old, jnp.max(s_t, axis=0))
                alpha = jnp.exp(m_old - m_new)
                p_t = jnp.exp(s_t - m_new)
                l_new = l_old * alpha + jnp.sum(p_t, axis=0)
                pv_t = lax.dot_general(v_h, p_t.astype(jnp.bfloat16),
                                       (((0,), (0,)), ((), ())),
                                       preferred_element_type=jnp.float32)
                ml_buf[cur, hd] = m_new
                ml_buf[cur, HQ + hd] = l_new
                o_buf[cur, pl.ds(hd * DH, DH), :] = o_old * alpha + pv_t
                return carry

            half = HQ // 2
            lax.fori_loop(0, half, head_step, 0)

            if h < N_DEV - 1:
                if h >= 1:
                    credit_rdma(nxt).wait_recv()
                early = []
                for src, dst, ssem, rsem in (
                    (o_buf.at[cur, pl.ds(0, half * DH)],
                     o_buf.at[nxt, pl.ds(0, half * DH)], o0_send, o0_recv),
                    (q_buf.at[cur], q_buf.at[nxt], q_send, q_recv),
                ):
                    r = pltpu.make_async_remote_copy(
                        src_ref=src, dst_ref=dst,
                        send_sem=ssem.at[cur], recv_sem=rsem.at[nxt],
                        device_id=(right,),
                        device_id_type=pl.DeviceIdType.MESH,
                    )
                    r.start()
                    early.append(r)

            lax.fori_loop(half, HQ, head_step, 0)

            if h < N_DEV - 1:
                late = []
                for src, dst, ssem, rsem in (
                    (o_buf.at[cur, pl.ds(half * DH, half * DH)],
                     o_buf.at[nxt, pl.ds(half * DH, half * DH)],
                     o1_send, o1_recv),
                    (ml_buf.at[cur], ml_buf.at[nxt], ml_send, ml_recv),
                ):
                    r = pltpu.make_async_remote_copy(
                        src_ref=src, dst_ref=dst,
                        send_sem=ssem.at[cur], recv_sem=rsem.at[nxt],
                        device_id=(right,),
                        device_id_type=pl.DeviceIdType.MESH,
                    )
                    r.start()
                    late.append(r)
                for r in early + late:
                    r.wait()
                if h < N_DEV - 2:
                    cr = credit_rdma(cur)
                    cr.start()
                    cr.wait_send()
            else:
                def norm_step(hd, carry):
                    o_buf[cur, pl.ds(hd * DH, DH), :] = (
                        o_buf[cur, pl.ds(hd * DH, DH), :] / ml_buf[cur, HQ + hd])
                    return carry

                lax.fori_loop(0, HQ, norm_step, 0)
                r = pltpu.make_async_remote_copy(
                    src_ref=o_buf.at[cur], dst_ref=y_dst,
                    send_sem=y_send, recv_sem=y_recv,
                    device_id=(right,), device_id_type=pl.DeviceIdType.MESH,
                )
                r.start()
                r.wait()
                out_ref[...] = y_dst[...]

    attn = pl.pallas_call(
        body,
        out_shape=jax.ShapeDtypeStruct((HQ * DH, sq), jnp.float32),
        in_specs=[pl.BlockSpec(memory_space=pltpu.VMEM)] * 3,
        out_specs=pl.BlockSpec(memory_space=pltpu.VMEM),
        scratch_shapes=[
            pltpu.VMEM((2, HQ * sq, DH), jnp.bfloat16),
            pltpu.VMEM((2, HQ * DH, sq), jnp.float32),
            pltpu.VMEM((2, 2 * HQ, sq), jnp.float32),
            pltpu.VMEM((HQ * DH, sq), jnp.float32),
            pltpu.VMEM((8, 128), jnp.float32),
            pltpu.VMEM((8, 128), jnp.float32),
            pltpu.SemaphoreType.DMA((2,)),
            pltpu.SemaphoreType.DMA((2,)),
            pltpu.SemaphoreType.DMA((2,)),
            pltpu.SemaphoreType.DMA((2,)),
            pltpu.SemaphoreType.DMA((2,)),
            pltpu.SemaphoreType.DMA((2,)),
            pltpu.SemaphoreType.DMA((2,)),
            pltpu.SemaphoreType.DMA((2,)),
            pltpu.SemaphoreType.DMA,
            pltpu.SemaphoreType.DMA,
            pltpu.SemaphoreType.DMA((2,)),
            pltpu.SemaphoreType.DMA((2,)),
        ],
        compiler_params=pltpu.CompilerParams(collective_id=0),
    )(q3, k3, v3)

    y = attn.reshape(HQ, DH, sq).transpose(2, 0, 1).reshape(sq, d)
    out = jnp.dot(y.astype(jnp.bfloat16), Wo.astype(jnp.bfloat16),
                  preferred_element_type=jnp.float32)
    return out.reshape(1, sq, d)
